# baseline (speedup 1.0000x reference)
import jax
import jax.numpy as jnp
from jax import lax
from jax.experimental import pallas as pl
from jax.experimental.pallas import tpu as pltpu
from jax.experimental.pallas import tpu_sc as plsc

NUM_USERS = 50000
NUM_ITEMS = 50000
N = NUM_USERS + NUM_ITEMS
H = 16
NC = 2
NS = 16

NODES_PER_TILE = 6272
NPAD = NS * NODES_PER_TILE
NCHUNK = 32
NODE_CHUNKS = NODES_PER_TILE // NCHUNK

E = 1600000
RPC = 4
EDGE_CHUNKS = 196
ROWS_PER_TILE = EDGE_CHUNKS * RPC
EP = NS * ROWS_PER_TILE * 128
GBYTES = 128 * H * 4

_F32 = jnp.float32
_I32 = jnp.int32


def _rsqrt16(d):
    i = lax.bitcast_convert_type(d, _I32)
    y = lax.bitcast_convert_type(jnp.int32(0x5F3759DF) - (i >> 1), _F32)
    half = d * 0.5
    for _ in range(3):
        y = y * (1.5 - half * y * y)
    return jnp.where(d > 0.5, y, jnp.zeros_like(y))


def _gcn_body(colp, rowp, x0p,
              s_out, y0, y1, y2, disx,
              acc, dacc,
              cbufs, rbufs, gbufs, onesb,
              zbuf, z1d, r1d, debuf, xbuf, abuf, sbuf,
              semia, semib, semga, semgb, semsa, semsb, semr, semw):
    c = lax.axis_index("c")
    t = lax.axis_index("s")
    coff = c * NPAD
    nbase0 = t * NODES_PER_TILE
    rbase0 = t * ROWS_PER_TILE
    semi = (semia, semib)
    semg = (semga, semgb)
    sems = (semsa, semsb)

    ones16 = jnp.ones((16,), _F32)
    zero16 = jnp.zeros((16,), _F32)

    @pl.loop(0, 8)
    def _c1(v):
        onesb[pl.ds(v * 16, 16)] = ones16

    @pl.loop(0, NCHUNK // 2)
    def _c2(n):
        zbuf[n, :] = zero16

    @pl.loop(0, NCHUNK // 16)
    def _c3(i):
        z1d[pl.ds(i * 16, 16)] = zero16

    @pl.loop(0, NODE_CHUNKS)
    def _zero(q):
        nb = nbase0 + q * NCHUNK
        pltpu.sync_copy(zbuf, acc.at[pl.ds(nb, NCHUNK // 2)])
        pltpu.sync_copy(zbuf, acc.at[pl.ds(nb + NCHUNK // 2, NCHUNK // 2)])
        pltpu.sync_copy(z1d, dacc.at[pl.ds(nb, NCHUNK)])

    plsc.subcore_barrier()

    max_rb = EP // 128 - RPC

    def idx_load(b, m, col_only):
        rb = jnp.minimum(rbase0 + m * RPC, max_rb)
        pltpu.async_copy(colp.at[pl.ds(rb, RPC)], cbufs[b], semi[b])
        if not col_only:
            pltpu.async_copy(rowp.at[pl.ds(rb, RPC)], rbufs[b], semi[b])

    def idx_wait(b, col_only):
        pltpu.make_async_copy(colp.at[pl.ds(0, RPC)], cbufs[b],
                              semi[b]).wait()
        if not col_only:
            pltpu.make_async_copy(rowp.at[pl.ds(0, RPC)], rbufs[b],
                                  semi[b]).wait()

    def add_coff(b):
        @pl.loop(0, RPC)
        def _off(r):
            @pl.loop(0, 8)
            def _offv(v):
                sl = pl.ds(v * 16, 16)
                cbufs[b][r, sl] = cbufs[b][r, sl] + coff


    @pl.loop(0, EDGE_CHUNKS // 2)
    def _deg(j):
        scats = {}
        for b in (0, 1):
            pltpu.sync_copy(colp.at[pl.ds(rbase0 + (2 * j + b) * RPC, RPC)],
                            cbufs[b])
            scats[b] = [
                pltpu.async_copy(onesb, dacc.at[cbufs[b].at[r]], sems[b],
                                 add=True)
                for r in range(RPC)
            ]
        for b in (0, 1):
            for d in scats[b]:
                d.wait()

    plsc.subcore_barrier()

    @pl.loop(0, NODE_CHUNKS)
    def _prep(q):
        nb = nbase0 + q * NCHUNK
        pltpu.sync_copy(dacc.at[pl.ds(nb, NCHUNK)], r1d)
        pltpu.sync_copy(x0p.at[pl.ds(nb * 32, NCHUNK * 32)], xbuf)

        @pl.loop(0, NCHUNK // 16)
        def _r(i):
            d = r1d[pl.ds(i * 16, 16)]
            r1d[pl.ds(i * 16, 16)] = _rsqrt16(d)

        @pl.loop(0, NCHUNK)
        def _b(n):
            idx = jnp.full((16,), n, _I32)
            debuf[n, :] = plsc.load_gather(r1d, [idx])

        @pl.loop(0, NCHUNK)
        def _y(n):
            xh = xbuf[pl.ds(n * 32 + c * H, H)]
            sbuf[n, :] = xh
            abuf[n, :] = debuf[n, :] * xh

        pltpu.sync_copy(debuf, disx.at[c, pl.ds(nb, NCHUNK)])
        pltpu.sync_copy(sbuf, s_out.at[pl.ds(coff + nb, NCHUNK)])
        pltpu.sync_copy(abuf, y0.at[pl.ds(coff + nb, NCHUNK)])

    plsc.subcore_barrier()

    for ysrc, ydst in [(y0, y1), (y1, y2), (y2, None)]:
        last = ydst is None

        @pl.loop(0, EDGE_CHUNKS // 2)
        def _edges(j):
            gaths = {}
            scats = {}
            for b in (0, 1):
                rb = rbase0 + (2 * j + b) * RPC
                pltpu.sync_copy(colp.at[pl.ds(rb, RPC)], cbufs[b])
                pltpu.sync_copy(rowp.at[pl.ds(rb, RPC)], rbufs[b])
                add_coff(b)
                gaths[b] = [
                    pltpu.async_copy(ysrc.at[cbufs[b].at[r]], gbufs[b].at[r],
                                     semg[b])
                    for r in range(RPC)
                ]
            for b in (0, 1):
                for d in gaths[b]:
                    d.wait()
                scats[b] = [
                    pltpu.async_copy(gbufs[b].at[r], acc.at[rbufs[b].at[r]],
                                     sems[b], add=True)
                    for r in range(RPC)
                ]
            for b in (0, 1):
                for d in scats[b]:
                    d.wait()

        plsc.subcore_barrier()

        @pl.loop(0, NODE_CHUNKS)
        def _post(q):
            nb = nbase0 + q * NCHUNK
            pltpu.sync_copy(acc.at[pl.ds(nb, NCHUNK)], abuf)
            pltpu.sync_copy(disx.at[c, pl.ds(nb, NCHUNK)], debuf)
            pltpu.sync_copy(s_out.at[pl.ds(coff + nb, NCHUNK)], sbuf)
            if not last:
                pltpu.sync_copy(zbuf, acc.at[pl.ds(nb, NCHUNK // 2)])
                pltpu.sync_copy(
                    zbuf, acc.at[pl.ds(nb + NCHUNK // 2, NCHUNK // 2)])

            @pl.loop(0, NCHUNK)
            def _n(n):
                d = debuf[n, :]
                x = d * abuf[n, :]
                if last:
                    sbuf[n, :] = (sbuf[n, :] + x) * 0.25
                else:
                    sbuf[n, :] = sbuf[n, :] + x
                    abuf[n, :] = d * x

            pltpu.sync_copy(sbuf, s_out.at[pl.ds(coff + nb, NCHUNK)])
            if not last:
                pltpu.sync_copy(abuf, ydst.at[pl.ds(coff + nb, NCHUNK)])

        plsc.subcore_barrier()


@jax.jit
def _lightgcn(colp, rowp, x0p):
    mesh = plsc.VectorSubcoreMesh(core_axis_name="c", subcore_axis_name="s",
                                  num_cores=NC, num_subcores=NS)
    f = pl.kernel(
        _gcn_body,
        out_type=(
            jax.ShapeDtypeStruct((2 * NPAD, H), _F32),
            jax.ShapeDtypeStruct((2 * NPAD, H), _F32),
            jax.ShapeDtypeStruct((2 * NPAD, H), _F32),
            jax.ShapeDtypeStruct((2 * NPAD, H), _F32),
            jax.ShapeDtypeStruct((NC, NPAD, H), _F32),
        ),
        mesh=mesh,
        compiler_params=pltpu.CompilerParams(needs_layout_passes=False,
                                             use_tc_tiling_on_sc=False),
        scratch_types=[
            pltpu.VMEM_SHARED((NPAD, H), _F32),
            pltpu.VMEM_SHARED((NPAD,), _F32),
            [pltpu.VMEM((RPC, 128), _I32)] * 2,
            [pltpu.VMEM((RPC, 128), _I32)] * 2,
            [pltpu.VMEM((RPC, 128, H), _F32)] * 2,
            pltpu.VMEM((128,), _F32),
            pltpu.VMEM((NCHUNK // 2, H), _F32),
            pltpu.VMEM((NCHUNK,), _F32),
            pltpu.VMEM((NCHUNK,), _F32),
            pltpu.VMEM((NCHUNK, H), _F32),
            pltpu.VMEM((NCHUNK * 32,), _F32),
            pltpu.VMEM((NCHUNK, H), _F32),
            pltpu.VMEM((NCHUNK, H), _F32),
            pltpu.SemaphoreType.DMA,
            pltpu.SemaphoreType.DMA,
            pltpu.SemaphoreType.DMA,
            pltpu.SemaphoreType.DMA,
            pltpu.SemaphoreType.DMA,
            pltpu.SemaphoreType.DMA,
            pltpu.SemaphoreType.DMA,
            pltpu.SemaphoreType.DMA,
        ],
    )
    return f(colp, rowp, x0p)


def kernel(edge_index, user_weight, item_weight):
    ei = edge_index.astype(_I32)
    pad = N + (jnp.arange(EP - E, dtype=_I32) % 16)
    rowp = jnp.concatenate([ei[0], pad]).reshape(EP // 128, 128)
    colp = jnp.concatenate([ei[1], pad]).reshape(EP // 128, 128)
    x0 = jnp.concatenate([user_weight, item_weight], axis=0)
    x0p = jnp.concatenate(
        [x0, jnp.zeros((NPAD - N, 32), _F32)], axis=0).reshape(NPAD * 32)
    s, _, _, _, _ = _lightgcn(colp, rowp, x0p)
    final = jnp.concatenate([s[:N], s[NPAD:NPAD + N]], axis=1)
    return final[:NUM_USERS], final[NUM_USERS:]


# speedup vs baseline: 1.0004x; 1.0004x over previous
import jax
import jax.numpy as jnp
from jax import lax
from jax.experimental import pallas as pl
from jax.experimental.pallas import tpu as pltpu
from jax.experimental.pallas import tpu_sc as plsc

NUM_USERS = 50000
NUM_ITEMS = 50000
N = NUM_USERS + NUM_ITEMS
H = 16
NC = 2
NS = 16

NODES_PER_TILE = 6272
NPAD = NS * NODES_PER_TILE
NCHUNK = 32
NODE_CHUNKS = NODES_PER_TILE // NCHUNK

E = 1600000
RPC = 4
EDGE_CHUNKS = 196
ROWS_PER_TILE = EDGE_CHUNKS * RPC
EP = NS * ROWS_PER_TILE * 128
GBYTES = 128 * H * 4

_F32 = jnp.float32
_I32 = jnp.int32


def _rsqrt16(d):
    i = lax.bitcast_convert_type(d, _I32)
    y = lax.bitcast_convert_type(jnp.int32(0x5F3759DF) - (i >> 1), _F32)
    half = d * 0.5
    for _ in range(3):
        y = y * (1.5 - half * y * y)
    return jnp.where(d > 0.5, y, jnp.zeros_like(y))


def _gcn_body(colp, rowp, x0p,
              s_out, y0, y1, y2, disx,
              acc, dacc,
              cbufs, rbufs, gbufs, onesb,
              zbuf, z1d, r1d, debuf, xbuf, abuf, sbuf,
              semia, semib, semga, semgb, semsa, semsb, semr, semw):
    c = lax.axis_index("c")
    t = lax.axis_index("s")
    coff = c * NPAD
    nbase0 = t * NODES_PER_TILE
    rbase0 = t * ROWS_PER_TILE
    semi = (semia, semib)
    semg = (semga, semgb)
    sems = (semsa, semsb)

    ones16 = jnp.ones((16,), _F32)
    zero16 = jnp.zeros((16,), _F32)

    @pl.loop(0, 8)
    def _c1(v):
        onesb[pl.ds(v * 16, 16)] = ones16

    @pl.loop(0, NCHUNK // 2)
    def _c2(n):
        zbuf[n, :] = zero16

    @pl.loop(0, NCHUNK // 16)
    def _c3(i):
        z1d[pl.ds(i * 16, 16)] = zero16

    @pl.loop(0, NODE_CHUNKS)
    def _zero(q):
        nb = nbase0 + q * NCHUNK
        pltpu.sync_copy(zbuf, acc.at[pl.ds(nb, NCHUNK // 2)])
        pltpu.sync_copy(zbuf, acc.at[pl.ds(nb + NCHUNK // 2, NCHUNK // 2)])
        pltpu.sync_copy(z1d, dacc.at[pl.ds(nb, NCHUNK)])

    plsc.subcore_barrier()

    max_rb = EP // 128 - RPC

    def idx_load(b, m, col_only):
        rb = jnp.minimum(rbase0 + m * RPC, max_rb)
        pltpu.async_copy(colp.at[pl.ds(rb, RPC)], cbufs[b], semi[b])
        if not col_only:
            pltpu.async_copy(rowp.at[pl.ds(rb, RPC)], rbufs[b], semi[b])

    def idx_wait(b, col_only):
        pltpu.make_async_copy(colp.at[pl.ds(0, RPC)], cbufs[b],
                              semi[b]).wait()
        if not col_only:
            pltpu.make_async_copy(rowp.at[pl.ds(0, RPC)], rbufs[b],
                                  semi[b]).wait()

    def add_coff(b):
        @pl.loop(0, RPC)
        def _off(r):
            @pl.loop(0, 8)
            def _offv(v):
                sl = pl.ds(v * 16, 16)
                cbufs[b][r, sl] = cbufs[b][r, sl] + coff


    with jax.named_scope("ph_deg"):
        @pl.loop(0, EDGE_CHUNKS // 2)
        def _deg(j):
            scats = {}
            for b in (0, 1):
                pltpu.sync_copy(
                    colp.at[pl.ds(rbase0 + (2 * j + b) * RPC, RPC)],
                    cbufs[b])
                scats[b] = [
                    pltpu.async_copy(onesb, dacc.at[cbufs[b].at[r]],
                                     sems[b], add=True)
                    for r in range(RPC)
                ]
            for b in (0, 1):
                for d in scats[b]:
                    d.wait()

    plsc.subcore_barrier()

    with jax.named_scope("ph_prep"):
      @pl.loop(0, NODE_CHUNKS)
      def _prep(q):
          nb = nbase0 + q * NCHUNK
          pltpu.sync_copy(dacc.at[pl.ds(nb, NCHUNK)], r1d)
          pltpu.sync_copy(x0p.at[pl.ds(nb * 32, NCHUNK * 32)], xbuf)

          @pl.loop(0, NCHUNK // 16)
          def _r(i):
              d = r1d[pl.ds(i * 16, 16)]
              r1d[pl.ds(i * 16, 16)] = _rsqrt16(d)

          @pl.loop(0, NCHUNK)
          def _b(n):
              idx = jnp.full((16,), n, _I32)
              debuf[n, :] = plsc.load_gather(r1d, [idx])

          @pl.loop(0, NCHUNK)
          def _y(n):
              xh = xbuf[pl.ds(n * 32 + c * H, H)]
              sbuf[n, :] = xh
              abuf[n, :] = debuf[n, :] * xh

          pltpu.sync_copy(debuf, disx.at[c, pl.ds(nb, NCHUNK)])
          pltpu.sync_copy(sbuf, s_out.at[pl.ds(coff + nb, NCHUNK)])
          pltpu.sync_copy(abuf, y0.at[pl.ds(coff + nb, NCHUNK)])

    plsc.subcore_barrier()

    for li, (ysrc, ydst) in enumerate([(y0, y1), (y1, y2), (y2, None)]):
        last = ydst is None

        with jax.named_scope(f"ph_edges{li}"):
          @pl.loop(0, EDGE_CHUNKS // 2)
          def _edges(j):
              gaths = {}
              scats = {}
              for b in (0, 1):
                  rb = rbase0 + (2 * j + b) * RPC
                  pltpu.sync_copy(colp.at[pl.ds(rb, RPC)], cbufs[b])
                  pltpu.sync_copy(rowp.at[pl.ds(rb, RPC)], rbufs[b])
                  add_coff(b)
                  gaths[b] = [
                      pltpu.async_copy(ysrc.at[cbufs[b].at[r]], gbufs[b].at[r],
                                       semg[b])
                      for r in range(RPC)
                  ]
              for b in (0, 1):
                  for d in gaths[b]:
                      d.wait()
                  scats[b] = [
                      pltpu.async_copy(gbufs[b].at[r], acc.at[rbufs[b].at[r]],
                                       sems[b], add=True)
                      for r in range(RPC)
                  ]
              for b in (0, 1):
                  for d in scats[b]:
                      d.wait()

        plsc.subcore_barrier()

        with jax.named_scope(f"ph_post{li}"):
          @pl.loop(0, NODE_CHUNKS)
          def _post(q):
              nb = nbase0 + q * NCHUNK
              pltpu.sync_copy(acc.at[pl.ds(nb, NCHUNK)], abuf)
              pltpu.sync_copy(disx.at[c, pl.ds(nb, NCHUNK)], debuf)
              pltpu.sync_copy(s_out.at[pl.ds(coff + nb, NCHUNK)], sbuf)
              if not last:
                  pltpu.sync_copy(zbuf, acc.at[pl.ds(nb, NCHUNK // 2)])
                  pltpu.sync_copy(
                      zbuf, acc.at[pl.ds(nb + NCHUNK // 2, NCHUNK // 2)])

              @pl.loop(0, NCHUNK)
              def _n(n):
                  d = debuf[n, :]
                  x = d * abuf[n, :]
                  if last:
                      sbuf[n, :] = (sbuf[n, :] + x) * 0.25
                  else:
                      sbuf[n, :] = sbuf[n, :] + x
                      abuf[n, :] = d * x

              pltpu.sync_copy(sbuf, s_out.at[pl.ds(coff + nb, NCHUNK)])
              if not last:
                  pltpu.sync_copy(abuf, ydst.at[pl.ds(coff + nb, NCHUNK)])

        plsc.subcore_barrier()


@jax.jit
def _lightgcn(colp, rowp, x0p):
    mesh = plsc.VectorSubcoreMesh(core_axis_name="c", subcore_axis_name="s",
                                  num_cores=NC, num_subcores=NS)
    f = pl.kernel(
        _gcn_body,
        out_type=(
            jax.ShapeDtypeStruct((2 * NPAD, H), _F32),
            jax.ShapeDtypeStruct((2 * NPAD, H), _F32),
            jax.ShapeDtypeStruct((2 * NPAD, H), _F32),
            jax.ShapeDtypeStruct((2 * NPAD, H), _F32),
            jax.ShapeDtypeStruct((NC, NPAD, H), _F32),
        ),
        mesh=mesh,
        compiler_params=pltpu.CompilerParams(needs_layout_passes=False,
                                             use_tc_tiling_on_sc=False),
        scratch_types=[
            pltpu.VMEM_SHARED((NPAD, H), _F32),
            pltpu.VMEM_SHARED((NPAD,), _F32),
            [pltpu.VMEM((RPC, 128), _I32)] * 2,
            [pltpu.VMEM((RPC, 128), _I32)] * 2,
            [pltpu.VMEM((RPC, 128, H), _F32)] * 2,
            pltpu.VMEM((128,), _F32),
            pltpu.VMEM((NCHUNK // 2, H), _F32),
            pltpu.VMEM((NCHUNK,), _F32),
            pltpu.VMEM((NCHUNK,), _F32),
            pltpu.VMEM((NCHUNK, H), _F32),
            pltpu.VMEM((NCHUNK * 32,), _F32),
            pltpu.VMEM((NCHUNK, H), _F32),
            pltpu.VMEM((NCHUNK, H), _F32),
            pltpu.SemaphoreType.DMA,
            pltpu.SemaphoreType.DMA,
            pltpu.SemaphoreType.DMA,
            pltpu.SemaphoreType.DMA,
            pltpu.SemaphoreType.DMA,
            pltpu.SemaphoreType.DMA,
            pltpu.SemaphoreType.DMA,
            pltpu.SemaphoreType.DMA,
        ],
    )
    return f(colp, rowp, x0p)


def kernel(edge_index, user_weight, item_weight):
    ei = edge_index.astype(_I32)
    pad = N + (jnp.arange(EP - E, dtype=_I32) % 16)
    rowp = jnp.concatenate([ei[0], pad]).reshape(EP // 128, 128)
    colp = jnp.concatenate([ei[1], pad]).reshape(EP // 128, 128)
    x0 = jnp.concatenate([user_weight, item_weight], axis=0)
    x0p = jnp.concatenate(
        [x0, jnp.zeros((NPAD - N, 32), _F32)], axis=0).reshape(NPAD * 32)
    s, _, _, _, _ = _lightgcn(colp, rowp, x0p)
    final = jnp.concatenate([s[:N], s[NPAD:NPAD + N]], axis=1)
    return final[:NUM_USERS], final[NUM_USERS:]


# speedup vs baseline: 1.3021x; 1.3015x over previous
import jax
import jax.numpy as jnp
from jax import lax
from jax.experimental import pallas as pl
from jax.experimental.pallas import tpu as pltpu
from jax.experimental.pallas import tpu_sc as plsc

NUM_USERS = 50000
NUM_ITEMS = 50000
N = NUM_USERS + NUM_ITEMS
H = 16
NC = 2
NS = 16

NODES_PER_TILE = 6272
NPAD = NS * NODES_PER_TILE
NCHUNK = 128
NODE_CHUNKS = NODES_PER_TILE // NCHUNK

E = 1600000
RPC = 4
EDGE_CHUNKS = 196
ROWS_PER_TILE = EDGE_CHUNKS * RPC
EP = NS * ROWS_PER_TILE * 128

_F32 = jnp.float32
_I32 = jnp.int32


def _rsqrt16(d):
    i = lax.bitcast_convert_type(d, _I32)
    y = lax.bitcast_convert_type(jnp.int32(0x5F3759DF) - (i >> 1), _F32)
    half = d * 0.5
    for _ in range(3):
        y = y * (1.5 - half * y * y)
    return jnp.where(d > 0.5, y, jnp.zeros_like(y))


def _gcn_body(colp, rowp, x0h,
              s_out, y0, y1, y2, disx,
              acc, dacc,
              cbufs, rbufs, gbufs, onesb, z1d, r1d,
              semga, semgb, semsa, semsb):
    c = lax.axis_index("c")
    t = lax.axis_index("s")
    coff = c * NPAD
    nbase0 = t * NODES_PER_TILE
    rbase0 = t * ROWS_PER_TILE
    semg = (semga, semgb)
    sems = (semsa, semsb)

    abuf = gbufs[0].at[0]
    debuf = gbufs[0].at[1]
    sbuf = gbufs[0].at[2]
    zbuf = gbufs[0].at[3]
    xbuf = gbufs[1].at[0]

    ones16 = jnp.ones((16,), _F32)
    zero16 = jnp.zeros((16,), _F32)

    @pl.loop(0, 8)
    def _c1(v):
        onesb[pl.ds(v * 16, 16)] = ones16

    @pl.loop(0, NCHUNK // 16)
    def _c3(i):
        z1d[pl.ds(i * 16, 16)] = zero16

    def fill_zbuf():
        @pl.loop(0, NCHUNK)
        def _z(n):
            zbuf[n, :] = zero16

    fill_zbuf()

    @pl.loop(0, NODE_CHUNKS)
    def _zero(q):
        nb = nbase0 + q * NCHUNK
        pltpu.sync_copy(zbuf, acc.at[pl.ds(nb, NCHUNK)])
        pltpu.sync_copy(z1d, dacc.at[pl.ds(nb, NCHUNK)])

    plsc.subcore_barrier()

    with jax.named_scope("ph_deg"):
        @pl.loop(0, EDGE_CHUNKS // 2)
        def _deg(j):
            scats = {}
            for b in (0, 1):
                pltpu.sync_copy(
                    colp.at[pl.ds(rbase0 + (2 * j + b) * RPC, RPC)],
                    cbufs[b])
                scats[b] = [
                    pltpu.async_copy(onesb, dacc.at[cbufs[b].at[r]],
                                     sems[b], add=True)
                    for r in range(RPC)
                ]
            for b in (0, 1):
                for d in scats[b]:
                    d.wait()

    plsc.subcore_barrier()

    with jax.named_scope("ph_prep"):
        @pl.loop(0, NODE_CHUNKS)
        def _prep(q):
            nb = nbase0 + q * NCHUNK
            pltpu.sync_copy(dacc.at[pl.ds(nb, NCHUNK)], r1d)
            pltpu.sync_copy(x0h.at[c, pl.ds(nb, NCHUNK)], xbuf)

            @pl.loop(0, NCHUNK // 16)
            def _r(i):
                d = r1d[pl.ds(i * 16, 16)]
                r1d[pl.ds(i * 16, 16)] = _rsqrt16(d)

            @pl.loop(0, NCHUNK)
            def _b(n):
                idx = jnp.full((16,), n, _I32)
                de = plsc.load_gather(r1d, [idx])
                debuf[n, :] = de
                xh = xbuf[n, :]
                sbuf[n, :] = xh
                xbuf[n, :] = de * xh

            pltpu.sync_copy(debuf, disx.at[c, pl.ds(nb, NCHUNK)])
            pltpu.sync_copy(sbuf, s_out.at[pl.ds(coff + nb, NCHUNK)])
            pltpu.sync_copy(xbuf, y0.at[pl.ds(coff + nb, NCHUNK)])

    plsc.subcore_barrier()

    for li, (ysrc, ydst) in enumerate([(y0, y1), (y1, y2), (y2, None)]):
        last = ydst is None

        with jax.named_scope(f"ph_edges{li}"):
            @pl.loop(0, EDGE_CHUNKS // 2)
            def _edges(j):
                gaths = {}
                scats = {}
                for b in (0, 1):
                    rb = rbase0 + (2 * j + b) * RPC
                    pltpu.sync_copy(colp.at[pl.ds(rb, RPC)], cbufs[b])
                    pltpu.sync_copy(rowp.at[pl.ds(rb, RPC)], rbufs[b])

                    @pl.loop(0, RPC)
                    def _off(r):
                        @pl.loop(0, 8)
                        def _offv(v):
                            sl = pl.ds(v * 16, 16)
                            cbufs[b][r, sl] = cbufs[b][r, sl] + coff

                    gaths[b] = [
                        pltpu.async_copy(ysrc.at[cbufs[b].at[r]],
                                         gbufs[b].at[r], semg[b])
                        for r in range(RPC)
                    ]
                for b in (0, 1):
                    for d in gaths[b]:
                        d.wait()
                    scats[b] = [
                        pltpu.async_copy(gbufs[b].at[r],
                                         acc.at[rbufs[b].at[r]],
                                         sems[b], add=True)
                        for r in range(RPC)
                    ]
                for b in (0, 1):
                    for d in scats[b]:
                        d.wait()

        plsc.subcore_barrier()

        with jax.named_scope(f"ph_post{li}"):
            fill_zbuf()

            @pl.loop(0, NODE_CHUNKS)
            def _post(q):
                nb = nbase0 + q * NCHUNK
                pltpu.sync_copy(acc.at[pl.ds(nb, NCHUNK)], abuf)
                pltpu.sync_copy(disx.at[c, pl.ds(nb, NCHUNK)], debuf)
                pltpu.sync_copy(s_out.at[pl.ds(coff + nb, NCHUNK)], sbuf)
                if not last:
                    pltpu.sync_copy(zbuf, acc.at[pl.ds(nb, NCHUNK)])

                @pl.loop(0, NCHUNK)
                def _n(n):
                    d = debuf[n, :]
                    x = d * abuf[n, :]
                    if last:
                        sbuf[n, :] = (sbuf[n, :] + x) * 0.25
                    else:
                        sbuf[n, :] = sbuf[n, :] + x
                        abuf[n, :] = d * x

                pltpu.sync_copy(sbuf, s_out.at[pl.ds(coff + nb, NCHUNK)])
                if not last:
                    pltpu.sync_copy(abuf,
                                    ydst.at[pl.ds(coff + nb, NCHUNK)])

        plsc.subcore_barrier()


@jax.jit
def _lightgcn(colp, rowp, x0h):
    mesh = plsc.VectorSubcoreMesh(core_axis_name="c", subcore_axis_name="s",
                                  num_cores=NC, num_subcores=NS)
    f = pl.kernel(
        _gcn_body,
        out_type=(
            jax.ShapeDtypeStruct((2 * NPAD, H), _F32),
            jax.ShapeDtypeStruct((2 * NPAD, H), _F32),
            jax.ShapeDtypeStruct((2 * NPAD, H), _F32),
            jax.ShapeDtypeStruct((2 * NPAD, H), _F32),
            jax.ShapeDtypeStruct((NC, NPAD, H), _F32),
        ),
        mesh=mesh,
        compiler_params=pltpu.CompilerParams(needs_layout_passes=False,
                                             use_tc_tiling_on_sc=False),
        scratch_types=[
            pltpu.VMEM_SHARED((NPAD, H), _F32),
            pltpu.VMEM_SHARED((NPAD,), _F32),
            [pltpu.VMEM((RPC, 128), _I32)] * 2,
            [pltpu.VMEM((RPC, 128), _I32)] * 2,
            [pltpu.VMEM((RPC, 128, H), _F32)] * 2,
            pltpu.VMEM((128,), _F32),
            pltpu.VMEM((NCHUNK,), _F32),
            pltpu.VMEM((NCHUNK,), _F32),
            pltpu.SemaphoreType.DMA,
            pltpu.SemaphoreType.DMA,
            pltpu.SemaphoreType.DMA,
            pltpu.SemaphoreType.DMA,
        ],
    )
    return f(colp, rowp, x0h)


def kernel(edge_index, user_weight, item_weight):
    ei = edge_index.astype(_I32)
    pad = N + (jnp.arange(EP - E, dtype=_I32) % 16)
    rowp = jnp.concatenate([ei[0], pad]).reshape(EP // 128, 128)
    colp = jnp.concatenate([ei[1], pad]).reshape(EP // 128, 128)
    x0 = jnp.concatenate([user_weight, item_weight], axis=0)
    x0p = jnp.concatenate([x0, jnp.zeros((NPAD - N, 32), _F32)], axis=0)
    x0h = jnp.stack([x0p[:, :H], x0p[:, H:]])
    s, _, _, _, _ = _lightgcn(colp, rowp, x0h)
    final = jnp.concatenate([s[:N], s[NPAD:NPAD + N]], axis=1)
    return final[:NUM_USERS], final[NUM_USERS:]


# speedup vs baseline: 1.6358x; 1.2563x over previous
import jax
import jax.numpy as jnp
from jax import lax
from jax.experimental import pallas as pl
from jax.experimental.pallas import tpu as pltpu
from jax.experimental.pallas import tpu_sc as plsc

NUM_USERS = 50000
NUM_ITEMS = 50000
N = NUM_USERS + NUM_ITEMS
H = 16
NC = 2
NS = 16

NODES_PER_TILE = 6272
NPAD = NS * NODES_PER_TILE
NCHUNK = 128
NODE_CHUNKS = NODES_PER_TILE // NCHUNK

E = 1600000
RPC = 4
EDGE_CHUNKS = 196
ROWS_PER_TILE = EDGE_CHUNKS * RPC
EP = NS * ROWS_PER_TILE * 128

_F32 = jnp.float32
_I32 = jnp.int32


def _rsqrt16(d):
    i = lax.bitcast_convert_type(d, _I32)
    y = lax.bitcast_convert_type(jnp.int32(0x5F3759DF) - (i >> 1), _F32)
    half = d * 0.5
    for _ in range(3):
        y = y * (1.5 - half * y * y)
    return jnp.where(d > 0.5, y, jnp.zeros_like(y))


def _gcn_body(colp, rowp, x0h,
              s_out, y0, y1, y2, disx,
              acc, dacc,
              cbufs, rbufs, gbufs, onesb, z1d, r1d,
              semga, semgb, semsa, semsb, semia, semib):
    c = lax.axis_index("c")
    t = lax.axis_index("s")
    coff = c * NPAD
    nbase0 = t * NODES_PER_TILE
    rbase0 = t * ROWS_PER_TILE
    semg = (semga, semgb)
    sems = (semsa, semsb)
    semi = (semia, semib)
    max_rb = EP // 128 - RPC

    def col_load(b, m):
        rb = jnp.minimum(rbase0 + m * RPC, max_rb)
        pltpu.async_copy(colp.at[pl.ds(rb, RPC)], cbufs[b], semi[b])

    def row_load(b, m):
        rb = jnp.minimum(rbase0 + m * RPC, max_rb)
        pltpu.async_copy(rowp.at[pl.ds(rb, RPC)], rbufs[b], semi[b])

    def col_wait(b):
        pltpu.make_async_copy(colp.at[pl.ds(0, RPC)], cbufs[b],
                              semi[b]).wait()

    def row_wait(b):
        pltpu.make_async_copy(rowp.at[pl.ds(0, RPC)], rbufs[b],
                              semi[b]).wait()

    abuf = gbufs[0].at[0]
    debuf = gbufs[0].at[1]
    sbuf = gbufs[0].at[2]
    zbuf = gbufs[0].at[3]
    xbuf = gbufs[1].at[0]

    ones16 = jnp.ones((16,), _F32)
    zero16 = jnp.zeros((16,), _F32)

    @pl.loop(0, 8)
    def _c1(v):
        onesb[pl.ds(v * 16, 16)] = ones16

    @pl.loop(0, NCHUNK // 16)
    def _c3(i):
        z1d[pl.ds(i * 16, 16)] = zero16

    def fill_zbuf():
        @pl.loop(0, NCHUNK)
        def _z(n):
            zbuf[n, :] = zero16

    fill_zbuf()

    @pl.loop(0, NODE_CHUNKS)
    def _zero(q):
        nb = nbase0 + q * NCHUNK
        pltpu.sync_copy(zbuf, acc.at[pl.ds(nb, NCHUNK)])
        pltpu.sync_copy(z1d, dacc.at[pl.ds(nb, NCHUNK)])

    plsc.subcore_barrier()

    with jax.named_scope("ph_deg"):
        col_load(0, 0)
        col_load(1, 1)

        @pl.loop(0, EDGE_CHUNKS // 2)
        def _deg(j):
            scats = {}
            for b in (0, 1):
                col_wait(b)
                scats[b] = [
                    pltpu.async_copy(onesb, dacc.at[cbufs[b].at[r]],
                                     sems[b], add=True)
                    for r in range(RPC)
                ]
            for b in (0, 1):
                for d in scats[b]:
                    d.wait()
                col_load(b, 2 * j + 2 + b)

        col_wait(0)
        col_wait(1)

    plsc.subcore_barrier()

    with jax.named_scope("ph_prep"):
        @pl.loop(0, NODE_CHUNKS)
        def _prep(q):
            nb = nbase0 + q * NCHUNK
            pltpu.sync_copy(dacc.at[pl.ds(nb, NCHUNK)], r1d)
            pltpu.sync_copy(x0h.at[c, pl.ds(nb, NCHUNK)], xbuf)

            @pl.loop(0, NCHUNK // 16)
            def _r(i):
                d = r1d[pl.ds(i * 16, 16)]
                r1d[pl.ds(i * 16, 16)] = _rsqrt16(d)

            @pl.loop(0, NCHUNK)
            def _b(n):
                idx = jnp.full((16,), n, _I32)
                de = plsc.load_gather(r1d, [idx])
                debuf[n, :] = de
                xh = xbuf[n, :]
                sbuf[n, :] = xh
                xbuf[n, :] = de * xh

            pltpu.sync_copy(debuf, disx.at[c, pl.ds(nb, NCHUNK)])
            pltpu.sync_copy(sbuf, s_out.at[pl.ds(coff + nb, NCHUNK)])
            pltpu.sync_copy(xbuf, y0.at[pl.ds(coff + nb, NCHUNK)])

    plsc.subcore_barrier()

    for li, (ysrc, ydst) in enumerate([(y0, y1), (y1, y2), (y2, None)]):
        last = ydst is None

        with jax.named_scope(f"ph_edges{li}"):
            col_load(0, 0)
            row_load(0, 0)
            col_load(1, 1)
            row_load(1, 1)

            @pl.loop(0, EDGE_CHUNKS // 2)
            def _edges(j):
                gaths = {}
                scats = {}
                for b in (0, 1):
                    col_wait(b)
                    row_wait(b)

                    @pl.loop(0, RPC)
                    def _off(r):
                        @pl.loop(0, 8)
                        def _offv(v):
                            sl = pl.ds(v * 16, 16)
                            cbufs[b][r, sl] = cbufs[b][r, sl] + coff

                    gaths[b] = [
                        pltpu.async_copy(ysrc.at[cbufs[b].at[r]],
                                         gbufs[b].at[r], semg[b])
                        for r in range(RPC)
                    ]
                for b in (0, 1):
                    for d in gaths[b]:
                        d.wait()
                    scats[b] = [
                        pltpu.async_copy(gbufs[b].at[r],
                                         acc.at[rbufs[b].at[r]],
                                         sems[b], add=True)
                        for r in range(RPC)
                    ]
                    col_load(b, 2 * j + 2 + b)
                for b in (0, 1):
                    for d in scats[b]:
                        d.wait()
                    row_load(b, 2 * j + 2 + b)

            for b in (0, 1):
                col_wait(b)
                row_wait(b)

        plsc.subcore_barrier()

        with jax.named_scope(f"ph_post{li}"):
            fill_zbuf()

            @pl.loop(0, NODE_CHUNKS)
            def _post(q):
                nb = nbase0 + q * NCHUNK
                pltpu.sync_copy(acc.at[pl.ds(nb, NCHUNK)], abuf)
                pltpu.sync_copy(disx.at[c, pl.ds(nb, NCHUNK)], debuf)
                pltpu.sync_copy(s_out.at[pl.ds(coff + nb, NCHUNK)], sbuf)
                if not last:
                    pltpu.sync_copy(zbuf, acc.at[pl.ds(nb, NCHUNK)])

                @pl.loop(0, NCHUNK)
                def _n(n):
                    d = debuf[n, :]
                    x = d * abuf[n, :]
                    if last:
                        sbuf[n, :] = (sbuf[n, :] + x) * 0.25
                    else:
                        sbuf[n, :] = sbuf[n, :] + x
                        abuf[n, :] = d * x

                pltpu.sync_copy(sbuf, s_out.at[pl.ds(coff + nb, NCHUNK)])
                if not last:
                    pltpu.sync_copy(abuf,
                                    ydst.at[pl.ds(coff + nb, NCHUNK)])

        plsc.subcore_barrier()


@jax.jit
def _lightgcn(colp, rowp, x0h):
    mesh = plsc.VectorSubcoreMesh(core_axis_name="c", subcore_axis_name="s",
                                  num_cores=NC, num_subcores=NS)
    f = pl.kernel(
        _gcn_body,
        out_type=(
            jax.ShapeDtypeStruct((2 * NPAD, H), _F32),
            jax.ShapeDtypeStruct((2 * NPAD, H), _F32),
            jax.ShapeDtypeStruct((2 * NPAD, H), _F32),
            jax.ShapeDtypeStruct((2 * NPAD, H), _F32),
            jax.ShapeDtypeStruct((NC, NPAD, H), _F32),
        ),
        mesh=mesh,
        compiler_params=pltpu.CompilerParams(needs_layout_passes=False,
                                             use_tc_tiling_on_sc=False),
        scratch_types=[
            pltpu.VMEM_SHARED((NPAD, H), _F32),
            pltpu.VMEM_SHARED((NPAD,), _F32),
            [pltpu.VMEM((RPC, 128), _I32)] * 2,
            [pltpu.VMEM((RPC, 128), _I32)] * 2,
            [pltpu.VMEM((RPC, 128, H), _F32)] * 2,
            pltpu.VMEM((128,), _F32),
            pltpu.VMEM((NCHUNK,), _F32),
            pltpu.VMEM((NCHUNK,), _F32),
            pltpu.SemaphoreType.DMA,
            pltpu.SemaphoreType.DMA,
            pltpu.SemaphoreType.DMA,
            pltpu.SemaphoreType.DMA,
            pltpu.SemaphoreType.DMA,
            pltpu.SemaphoreType.DMA,
        ],
    )
    return f(colp, rowp, x0h)


def kernel(edge_index, user_weight, item_weight):
    ei = edge_index.astype(_I32)
    pad = N + (jnp.arange(EP - E, dtype=_I32) % 16)
    rowp = jnp.concatenate([ei[0], pad]).reshape(EP // 128, 128)
    colp = jnp.concatenate([ei[1], pad]).reshape(EP // 128, 128)
    x0 = jnp.concatenate([user_weight, item_weight], axis=0)
    x0p = jnp.concatenate([x0, jnp.zeros((NPAD - N, 32), _F32)], axis=0)
    x0h = jnp.stack([x0p[:, :H], x0p[:, H:]])
    s, _, _, _, _ = _lightgcn(colp, rowp, x0h)
    final = jnp.concatenate([s[:N], s[NPAD:NPAD + N]], axis=1)
    return final[:NUM_USERS], final[NUM_USERS:]


# speedup vs baseline: 1.7781x; 1.0870x over previous
import jax
import jax.numpy as jnp
from jax import lax
from jax.experimental import pallas as pl
from jax.experimental.pallas import tpu as pltpu
from jax.experimental.pallas import tpu_sc as plsc

NUM_USERS = 50000
NUM_ITEMS = 50000
N = NUM_USERS + NUM_ITEMS
H = 16
NC = 2
NS = 16

NODES_PER_TILE = 6272
NPAD = NS * NODES_PER_TILE
NCHUNK = 128
NODE_CHUNKS = NODES_PER_TILE // NCHUNK

E = 1600000
RPC = 4
EDGE_CHUNKS = 196
ROWS_PER_TILE = EDGE_CHUNKS * RPC
EP = NS * ROWS_PER_TILE * 128

_F32 = jnp.float32
_I32 = jnp.int32


def _rsqrt16(d):
    i = lax.bitcast_convert_type(d, _I32)
    y = lax.bitcast_convert_type(jnp.int32(0x5F3759DF) - (i >> 1), _F32)
    half = d * 0.5
    for _ in range(3):
        y = y * (1.5 - half * y * y)
    return jnp.where(d > 0.5, y, jnp.zeros_like(y))


def _gcn_body(colp, rowp, x0h,
              s_out, y0, y1, y2,
              acc, dacc,
              cbufs, rbufs, gbufs, onesb, z1d, r1d,
              semga, semgb, semsa, semsb, semia, semib, semra, semrb,
              semr, semw):
    c = lax.axis_index("c")
    t = lax.axis_index("s")
    coff = c * NPAD
    nbase0 = t * NODES_PER_TILE
    rbase0 = t * ROWS_PER_TILE
    semg = (semga, semgb)
    sems = (semsa, semsb)
    semic = (semia, semib)
    semir = (semra, semrb)
    max_rb = EP // 128 - RPC

    def col_load(b, m):
        rb = jnp.minimum(rbase0 + m * RPC, max_rb)
        pltpu.async_copy(colp.at[pl.ds(rb, RPC)], cbufs[b], semic[b])

    def row_load(b, m):
        rb = jnp.minimum(rbase0 + m * RPC, max_rb)
        pltpu.async_copy(rowp.at[pl.ds(rb, RPC)], rbufs[b], semir[b])

    def col_wait(b):
        pltpu.make_async_copy(colp.at[pl.ds(0, RPC)], cbufs[b],
                              semic[b]).wait()

    def row_wait(b):
        pltpu.make_async_copy(rowp.at[pl.ds(0, RPC)], rbufs[b],
                              semir[b]).wait()

    def fire_gaths(b, ysrc):
        @pl.loop(0, RPC)
        def _off(r):
            @pl.loop(0, 8)
            def _offv(v):
                sl = pl.ds(v * 16, 16)
                cbufs[b][r, sl] = cbufs[b][r, sl] + coff

        return [
            pltpu.async_copy(ysrc.at[cbufs[b].at[r]], gbufs[b].at[r],
                             semg[b])
            for r in range(RPC)
        ]

    def wait_gaths(b, ysrc):
        for r in range(RPC):
            pltpu.make_async_copy(ysrc.at[cbufs[b].at[r]], gbufs[b].at[r],
                                  semg[b]).wait()

    def fire_scats(b):
        return [
            pltpu.async_copy(gbufs[b].at[r], acc.at[rbufs[b].at[r]],
                             sems[b], add=True)
            for r in range(RPC)
        ]

    def wait_scats(b):
        for r in range(RPC):
            pltpu.make_async_copy(gbufs[b].at[r], acc.at[rbufs[b].at[r]],
                                  sems[b]).wait()

    abuf = gbufs[0].at[0]
    sbuf = gbufs[0].at[2]
    zbuf = gbufs[0].at[3]
    xbuf = gbufs[1].at[0]

    ones16 = jnp.ones((16,), _F32)
    zero16 = jnp.zeros((16,), _F32)

    @pl.loop(0, 8)
    def _c1(v):
        onesb[pl.ds(v * 16, 16)] = ones16

    @pl.loop(0, NCHUNK // 16)
    def _c3(i):
        z1d[pl.ds(i * 16, 16)] = zero16

    def fill_zbuf():
        @pl.loop(0, NCHUNK)
        def _z(n):
            zbuf[n, :] = zero16

    fill_zbuf()

    @pl.loop(0, NODE_CHUNKS)
    def _zero(q):
        nb = nbase0 + q * NCHUNK
        pltpu.sync_copy(zbuf, acc.at[pl.ds(nb, NCHUNK)])
        pltpu.sync_copy(z1d, dacc.at[pl.ds(nb, NCHUNK)])

    plsc.subcore_barrier()

    with jax.named_scope("ph_deg"):
        col_load(0, 0)
        col_load(1, 1)

        @pl.loop(0, EDGE_CHUNKS // 2)
        def _deg(j):
            scats = {}
            for b in (0, 1):
                col_wait(b)
                scats[b] = [
                    pltpu.async_copy(onesb, dacc.at[cbufs[b].at[r]],
                                     sems[b], add=True)
                    for r in range(RPC)
                ]
            for b in (0, 1):
                for d in scats[b]:
                    d.wait()
                col_load(b, 2 * j + 2 + b)

        col_wait(0)
        col_wait(1)

    plsc.subcore_barrier()

    with jax.named_scope("ph_prep"):
        @pl.loop(0, NODE_CHUNKS)
        def _prep(q):
            nb = nbase0 + q * NCHUNK
            cp_x = pltpu.async_copy(x0h.at[c, pl.ds(nb, NCHUNK)], xbuf,
                                    semr)
            pltpu.sync_copy(dacc.at[pl.ds(nb, NCHUNK)], r1d)

            @pl.loop(0, NCHUNK // 16)
            def _r(i):
                d = r1d[pl.ds(i * 16, 16)]
                r1d[pl.ds(i * 16, 16)] = _rsqrt16(d)

            pltpu.sync_copy(r1d, dacc.at[pl.ds(nb, NCHUNK)])
            cp_x.wait()

            @pl.loop(0, NCHUNK)
            def _y(n):
                de = plsc.load_gather(r1d, [jnp.full((16,), n, _I32)])
                xh = xbuf[n, :]
                sbuf[n, :] = xh
                xbuf[n, :] = de * xh

            ws = [
                pltpu.async_copy(sbuf, s_out.at[pl.ds(coff + nb, NCHUNK)],
                                 semw),
                pltpu.async_copy(xbuf, y0.at[pl.ds(coff + nb, NCHUNK)],
                                 semw),
            ]
            for d in ws:
                d.wait()

    plsc.subcore_barrier()

    for li, (ysrc, ydst) in enumerate([(y0, y1), (y1, y2), (y2, None)]):
        last = ydst is None

        with jax.named_scope(f"ph_edges{li}"):
            for b in (0, 1):
                col_load(b, b)
                row_load(b, b)

            @pl.loop(0, EDGE_CHUNKS // 2)
            def _edges(j):
                for b in (0, 1):
                    col_wait(b)
                    row_wait(b)
                    fire_gaths(b, ysrc)
                for b in (0, 1):
                    wait_gaths(b, ysrc)
                    fire_scats(b)
                    col_load(b, 2 * j + 2 + b)
                for b in (0, 1):
                    wait_scats(b)
                    row_load(b, 2 * j + 2 + b)

            for b in (0, 1):
                col_wait(b)
                row_wait(b)

        plsc.subcore_barrier()

        with jax.named_scope(f"ph_post{li}"):
            fill_zbuf()

            @pl.loop(0, NODE_CHUNKS)
            def _post(q):
                nb = nbase0 + q * NCHUNK
                cp_s = pltpu.async_copy(
                    s_out.at[pl.ds(coff + nb, NCHUNK)], sbuf, semr)
                pltpu.sync_copy(acc.at[pl.ds(nb, NCHUNK)], abuf)
                pltpu.sync_copy(dacc.at[pl.ds(nb, NCHUNK)], r1d)
                if not last:
                    pltpu.sync_copy(zbuf, acc.at[pl.ds(nb, NCHUNK)])
                cp_s.wait()

                @pl.loop(0, NCHUNK)
                def _n(n):
                    de = plsc.load_gather(r1d, [jnp.full((16,), n, _I32)])
                    x = de * abuf[n, :]
                    if last:
                        sbuf[n, :] = (sbuf[n, :] + x) * 0.25
                    else:
                        sbuf[n, :] = sbuf[n, :] + x
                        abuf[n, :] = de * x

                ws = [pltpu.async_copy(
                    sbuf, s_out.at[pl.ds(coff + nb, NCHUNK)], semw)]
                if not last:
                    ws.append(pltpu.async_copy(
                        abuf, ydst.at[pl.ds(coff + nb, NCHUNK)], semw))
                for d in ws:
                    d.wait()

        plsc.subcore_barrier()


@jax.jit
def _lightgcn(colp, rowp, x0h):
    mesh = plsc.VectorSubcoreMesh(core_axis_name="c", subcore_axis_name="s",
                                  num_cores=NC, num_subcores=NS)
    f = pl.kernel(
        _gcn_body,
        out_type=(
            jax.ShapeDtypeStruct((2 * NPAD, H), _F32),
            jax.ShapeDtypeStruct((2 * NPAD, H), _F32),
            jax.ShapeDtypeStruct((2 * NPAD, H), _F32),
            jax.ShapeDtypeStruct((2 * NPAD, H), _F32),
        ),
        mesh=mesh,
        compiler_params=pltpu.CompilerParams(needs_layout_passes=False,
                                             use_tc_tiling_on_sc=False),
        scratch_types=[
            pltpu.VMEM_SHARED((NPAD, H), _F32),
            pltpu.VMEM_SHARED((NPAD,), _F32),
            [pltpu.VMEM((RPC, 128), _I32)] * 2,
            [pltpu.VMEM((RPC, 128), _I32)] * 2,
            [pltpu.VMEM((RPC, 128, H), _F32)] * 2,
            pltpu.VMEM((128,), _F32),
            pltpu.VMEM((NCHUNK,), _F32),
            pltpu.VMEM((NCHUNK,), _F32),
            pltpu.SemaphoreType.DMA,
            pltpu.SemaphoreType.DMA,
            pltpu.SemaphoreType.DMA,
            pltpu.SemaphoreType.DMA,
            pltpu.SemaphoreType.DMA,
            pltpu.SemaphoreType.DMA,
            pltpu.SemaphoreType.DMA,
            pltpu.SemaphoreType.DMA,
            pltpu.SemaphoreType.DMA,
            pltpu.SemaphoreType.DMA,
        ],
    )
    return f(colp, rowp, x0h)


def kernel(edge_index, user_weight, item_weight):
    ei = edge_index.astype(_I32)
    pad = N + (jnp.arange(EP - E, dtype=_I32) % 16)
    rowp = jnp.concatenate([ei[0], pad]).reshape(EP // 128, 128)
    colp = jnp.concatenate([ei[1], pad]).reshape(EP // 128, 128)
    x0 = jnp.concatenate([user_weight, item_weight], axis=0)
    x0p = jnp.concatenate([x0, jnp.zeros((NPAD - N, 32), _F32)], axis=0)
    x0h = jnp.stack([x0p[:, :H], x0p[:, H:]])
    s, _, _, _ = _lightgcn(colp, rowp, x0h)
    final = jnp.concatenate([s[:N], s[NPAD:NPAD + N]], axis=1)
    return final[:NUM_USERS], final[NUM_USERS:]


# speedup vs baseline: 1.7782x; 1.0000x over previous
import jax
import jax.numpy as jnp
from jax import lax
from jax.experimental import pallas as pl
from jax.experimental.pallas import tpu as pltpu
from jax.experimental.pallas import tpu_sc as plsc

NUM_USERS = 50000
NUM_ITEMS = 50000
N = NUM_USERS + NUM_ITEMS
H = 16
NC = 2
NS = 16

NODES_PER_TILE = 6272
NPAD = NS * NODES_PER_TILE
NCHUNK = 128
NODE_CHUNKS = NODES_PER_TILE // NCHUNK

E = 1600000
RPC = 4
EDGE_CHUNKS = 196
ROWS_PER_TILE = EDGE_CHUNKS * RPC
EP = NS * ROWS_PER_TILE * 128

_F32 = jnp.float32
_I32 = jnp.int32


def _rsqrt16(d):
    i = lax.bitcast_convert_type(d, _I32)
    y = lax.bitcast_convert_type(jnp.int32(0x5F3759DF) - (i >> 1), _F32)
    half = d * 0.5
    for _ in range(3):
        y = y * (1.5 - half * y * y)
    return jnp.where(d > 0.5, y, jnp.zeros_like(y))


def _gcn_body(colp, rowp, x0h,
              s_out, y0, y1, y2,
              acc, dacc,
              cbufs, rbufs, gbufs, onesb, z1d, r1d,
              semga, semgb, semsa, semsb, semia, semib, semra, semrb,
              semr, semw):
    c = lax.axis_index("c")
    t = lax.axis_index("s")
    coff = c * NPAD
    nbase0 = t * NODES_PER_TILE
    rbase0 = t * ROWS_PER_TILE
    semg = (semga, semgb)
    sems = (semsa, semsb)
    semic = (semia, semib)
    semir = (semra, semrb)
    max_rb = EP // 128 - RPC

    def col_load(b, m):
        rb = jnp.minimum(rbase0 + m * RPC, max_rb)
        pltpu.async_copy(colp.at[pl.ds(rb, RPC)], cbufs[b], semic[b])

    def row_load(b, m):
        rb = jnp.minimum(rbase0 + m * RPC, max_rb)
        pltpu.async_copy(rowp.at[pl.ds(rb, RPC)], rbufs[b], semir[b])

    def col_wait(b):
        pltpu.make_async_copy(colp.at[pl.ds(0, RPC)], cbufs[b],
                              semic[b]).wait()

    def row_wait(b):
        pltpu.make_async_copy(rowp.at[pl.ds(0, RPC)], rbufs[b],
                              semir[b]).wait()

    def fire_gaths(b, ysrc):
        @pl.loop(0, RPC)
        def _off(r):
            @pl.loop(0, 8)
            def _offv(v):
                sl = pl.ds(v * 16, 16)
                cbufs[b][r, sl] = cbufs[b][r, sl] + coff

        return [
            pltpu.async_copy(ysrc.at[cbufs[b].at[r]], gbufs[b].at[r],
                             semg[b])
            for r in range(RPC)
        ]

    def wait_gaths(b, ysrc):
        for r in range(RPC):
            pltpu.make_async_copy(ysrc.at[cbufs[b].at[r]], gbufs[b].at[r],
                                  semg[b]).wait()

    def fire_scats(b):
        return [
            pltpu.async_copy(gbufs[b].at[r], acc.at[rbufs[b].at[r]],
                             sems[b], add=True)
            for r in range(RPC)
        ]

    def wait_scats(b):
        for r in range(RPC):
            pltpu.make_async_copy(gbufs[b].at[r], acc.at[rbufs[b].at[r]],
                                  sems[b]).wait()

    abuf = gbufs[0].at[0]
    sbuf = gbufs[0].at[2]
    zbuf = gbufs[0].at[3]
    xbuf = gbufs[1].at[0]

    ones16 = jnp.ones((16,), _F32)
    zero16 = jnp.zeros((16,), _F32)

    @pl.loop(0, 8)
    def _c1(v):
        onesb[pl.ds(v * 16, 16)] = ones16

    @pl.loop(0, NCHUNK // 16)
    def _c3(i):
        z1d[pl.ds(i * 16, 16)] = zero16

    def fill_zbuf():
        @pl.loop(0, NCHUNK)
        def _z(n):
            zbuf[n, :] = zero16

    fill_zbuf()

    @pl.loop(0, NODE_CHUNKS)
    def _zero(q):
        nb = nbase0 + q * NCHUNK
        pltpu.sync_copy(zbuf, acc.at[pl.ds(nb, NCHUNK)])
        pltpu.sync_copy(z1d, dacc.at[pl.ds(nb, NCHUNK)])

    plsc.subcore_barrier()

    with jax.named_scope("ph_deg"):
        col_load(0, 0)
        col_load(1, 1)

        @pl.loop(0, EDGE_CHUNKS // 2)
        def _deg(j):
            scats = {}
            for b in (0, 1):
                col_wait(b)
                scats[b] = [
                    pltpu.async_copy(onesb, dacc.at[cbufs[b].at[r]],
                                     sems[b], add=True)
                    for r in range(RPC)
                ]
            for b in (0, 1):
                for d in scats[b]:
                    d.wait()
                col_load(b, 2 * j + 2 + b)

        col_wait(0)
        col_wait(1)

    plsc.subcore_barrier()

    def prep_body(q, first):
        nb = nbase0 + q * NCHUNK
        if not first:
            for _ in range(2):
                pltpu.make_async_copy(
                    sbuf, s_out.at[pl.ds(coff, NCHUNK)], semw).wait()
        cp_x = pltpu.async_copy(x0h.at[c, pl.ds(nb, NCHUNK)], xbuf, semr)
        pltpu.sync_copy(dacc.at[pl.ds(nb, NCHUNK)], r1d)

        @pl.loop(0, NCHUNK // 16)
        def _r(i):
            d = r1d[pl.ds(i * 16, 16)]
            r1d[pl.ds(i * 16, 16)] = _rsqrt16(d)

        pltpu.sync_copy(r1d, dacc.at[pl.ds(nb, NCHUNK)])
        cp_x.wait()

        @pl.loop(0, NCHUNK)
        def _y(n):
            de = plsc.load_gather(r1d, [jnp.full((16,), n, _I32)])
            xh = xbuf[n, :]
            sbuf[n, :] = xh
            xbuf[n, :] = de * xh

        pltpu.async_copy(sbuf, s_out.at[pl.ds(coff + nb, NCHUNK)], semw)
        pltpu.async_copy(xbuf, y0.at[pl.ds(coff + nb, NCHUNK)], semw)

    with jax.named_scope("ph_prep"):
        prep_body(0, True)

        @pl.loop(1, NODE_CHUNKS)
        def _prep(q):
            prep_body(q, False)

        for _ in range(2):
            pltpu.make_async_copy(sbuf, s_out.at[pl.ds(coff, NCHUNK)],
                                  semw).wait()

    plsc.subcore_barrier()

    for li, (ysrc, ydst) in enumerate([(y0, y1), (y1, y2), (y2, None)]):
        last = ydst is None

        with jax.named_scope(f"ph_edges{li}"):
            for b in (0, 1):
                col_load(b, b)
                row_load(b, b)

            @pl.loop(0, EDGE_CHUNKS // 2)
            def _edges(j):
                for b in (0, 1):
                    col_wait(b)
                    row_wait(b)
                    fire_gaths(b, ysrc)
                for b in (0, 1):
                    wait_gaths(b, ysrc)
                    fire_scats(b)
                    col_load(b, 2 * j + 2 + b)
                for b in (0, 1):
                    wait_scats(b)
                    row_load(b, 2 * j + 2 + b)

            for b in (0, 1):
                col_wait(b)
                row_wait(b)

        plsc.subcore_barrier()

        nw = 1 if last else 2

        def drain_post():
            for _ in range(nw):
                pltpu.make_async_copy(
                    sbuf, s_out.at[pl.ds(coff, NCHUNK)], semw).wait()

        def post_body(q, first, last=last, ydst=ydst):
            nb = nbase0 + q * NCHUNK
            if not first:
                drain_post()
            cp_s = pltpu.async_copy(
                s_out.at[pl.ds(coff + nb, NCHUNK)], sbuf, semr)
            pltpu.sync_copy(acc.at[pl.ds(nb, NCHUNK)], abuf)
            pltpu.sync_copy(dacc.at[pl.ds(nb, NCHUNK)], r1d)
            if not last:
                pltpu.sync_copy(zbuf, acc.at[pl.ds(nb, NCHUNK)])
            cp_s.wait()

            @pl.loop(0, NCHUNK)
            def _n(n):
                de = plsc.load_gather(r1d, [jnp.full((16,), n, _I32)])
                x = de * abuf[n, :]
                if last:
                    sbuf[n, :] = (sbuf[n, :] + x) * 0.25
                else:
                    sbuf[n, :] = sbuf[n, :] + x
                    abuf[n, :] = de * x

            pltpu.async_copy(sbuf, s_out.at[pl.ds(coff + nb, NCHUNK)],
                             semw)
            if not last:
                pltpu.async_copy(abuf, ydst.at[pl.ds(coff + nb, NCHUNK)],
                                 semw)

        with jax.named_scope(f"ph_post{li}"):
            fill_zbuf()
            post_body(0, True)

            @pl.loop(1, NODE_CHUNKS)
            def _post(q):
                post_body(q, False)

            drain_post()

        plsc.subcore_barrier()


@jax.jit
def _lightgcn(colp, rowp, x0h):
    mesh = plsc.VectorSubcoreMesh(core_axis_name="c", subcore_axis_name="s",
                                  num_cores=NC, num_subcores=NS)
    f = pl.kernel(
        _gcn_body,
        out_type=(
            jax.ShapeDtypeStruct((2 * NPAD, H), _F32),
            jax.ShapeDtypeStruct((2 * NPAD, H), _F32),
            jax.ShapeDtypeStruct((2 * NPAD, H), _F32),
            jax.ShapeDtypeStruct((2 * NPAD, H), _F32),
        ),
        mesh=mesh,
        compiler_params=pltpu.CompilerParams(needs_layout_passes=False,
                                             use_tc_tiling_on_sc=False),
        scratch_types=[
            pltpu.VMEM_SHARED((NPAD, H), _F32),
            pltpu.VMEM_SHARED((NPAD,), _F32),
            [pltpu.VMEM((RPC, 128), _I32)] * 2,
            [pltpu.VMEM((RPC, 128), _I32)] * 2,
            [pltpu.VMEM((RPC, 128, H), _F32)] * 2,
            pltpu.VMEM((128,), _F32),
            pltpu.VMEM((NCHUNK,), _F32),
            pltpu.VMEM((NCHUNK,), _F32),
            pltpu.SemaphoreType.DMA,
            pltpu.SemaphoreType.DMA,
            pltpu.SemaphoreType.DMA,
            pltpu.SemaphoreType.DMA,
            pltpu.SemaphoreType.DMA,
            pltpu.SemaphoreType.DMA,
            pltpu.SemaphoreType.DMA,
            pltpu.SemaphoreType.DMA,
            pltpu.SemaphoreType.DMA,
            pltpu.SemaphoreType.DMA,
        ],
    )
    return f(colp, rowp, x0h)


def kernel(edge_index, user_weight, item_weight):
    ei = edge_index.astype(_I32)
    pad = N + (jnp.arange(EP - E, dtype=_I32) % 16)
    rowp = jnp.concatenate([ei[0], pad]).reshape(EP // 128, 128)
    colp = jnp.concatenate([ei[1], pad]).reshape(EP // 128, 128)
    x0 = jnp.concatenate([user_weight, item_weight], axis=0)
    x0p = jnp.concatenate([x0, jnp.zeros((NPAD - N, 32), _F32)], axis=0)
    x0h = jnp.stack([x0p[:, :H], x0p[:, H:]])
    s, _, _, _ = _lightgcn(colp, rowp, x0h)
    final = jnp.concatenate([s[:N], s[NPAD:NPAD + N]], axis=1)
    return final[:NUM_USERS], final[NUM_USERS:]
